# Initial kernel scaffold; baseline (speedup 1.0000x reference)
import functools

import jax
import jax.numpy as jnp
from jax import lax
from jax.experimental import pallas as pl
from jax.experimental.pallas import tpu as pltpu
from jax.experimental.pallas import tpu_sc as plsc

G_GRAPHS = 64
NC = 2
NS = 16
NW = NC * NS
CHUNK = 128
ROWS_PER_TILE = 626
NPAD = NS * ROWS_PER_TILE

_F32 = jnp.float32


def _mesh():
    return plsc.VectorSubcoreMesh(core_axis_name="c", subcore_axis_name="s")


def _sc_degree(cols3, ones_col, zero_col):
    ch = cols3.shape[1]

    @functools.partial(
        pl.kernel,
        mesh=_mesh(),
        out_type=jax.ShapeDtypeStruct((NC, NPAD, 1), _F32),
        scratch_types=[
            pltpu.VMEM((ch, CHUNK), jnp.int32),
            pltpu.VMEM((CHUNK, 1), _F32),
        ],
    )
    def deg_k(col_hbm, ones_hbm, zero_hbm, out_hbm, col_v, ones_v):
        def inner(deg_sh):
            c = lax.axis_index("c")
            s = lax.axis_index("s")
            w = c * NS + s
            pltpu.sync_copy(col_hbm.at[w], col_v)
            pltpu.sync_copy(ones_hbm, ones_v)
            pltpu.sync_copy(zero_hbm, deg_sh.at[pl.ds(s * ROWS_PER_TILE, ROWS_PER_TILE)])
            plsc.subcore_barrier()

            def body(j, carry):
                pltpu.sync_copy(ones_v, deg_sh.at[col_v.at[j]], add=True)
                return carry

            lax.fori_loop(0, ch, body, 0)
            plsc.subcore_barrier()
            pltpu.sync_copy(
                deg_sh.at[pl.ds(s * ROWS_PER_TILE, ROWS_PER_TILE)],
                out_hbm.at[c, pl.ds(s * ROWS_PER_TILE, ROWS_PER_TILE)],
            )

        pl.run_scoped(inner, pltpu.VMEM_SHARED((NPAD, 1), _F32))

    return deg_k(cols3, ones_col, zero_col)


def _sc_aggregate(y, rows3, cols3, zero_blk):
    h = y.shape[1]
    ch = rows3.shape[1]

    @functools.partial(
        pl.kernel,
        mesh=_mesh(),
        out_type=jax.ShapeDtypeStruct((NC, NPAD, h), _F32),
        scratch_types=[
            pltpu.VMEM((ch, CHUNK), jnp.int32),
            pltpu.VMEM((ch, CHUNK), jnp.int32),
            pltpu.VMEM((CHUNK, h), _F32),
            pltpu.VMEM((CHUNK, h), _F32),
            pltpu.SemaphoreType.DMA,
            pltpu.SemaphoreType.DMA,
        ],
    )
    def agg_k(y_hbm, row_hbm, col_hbm, zero_hbm, out_hbm, row_v, col_v, g0, g1, s0, s1):
        def inner(z_sh):
            c = lax.axis_index("c")
            s = lax.axis_index("s")
            w = c * NS + s
            pltpu.sync_copy(row_hbm.at[w], row_v)
            pltpu.sync_copy(col_hbm.at[w], col_v)
            pltpu.sync_copy(zero_hbm, z_sh.at[pl.ds(s * ROWS_PER_TILE, ROWS_PER_TILE)])
            plsc.subcore_barrier()

            pltpu.async_copy(y_hbm.at[row_v.at[0]], g0, s0)
            pltpu.async_copy(y_hbm.at[row_v.at[1]], g1, s1)

            def body(i, carry):
                j0 = 2 * i
                j1 = j0 + 1
                pltpu.make_async_copy(y_hbm.at[row_v.at[j0]], g0, s0).wait()
                pltpu.sync_copy(g0, z_sh.at[col_v.at[j0]], add=True)

                @pl.when(j0 + 2 < ch)
                def _():
                    pltpu.async_copy(y_hbm.at[row_v.at[j0 + 2]], g0, s0)

                pltpu.make_async_copy(y_hbm.at[row_v.at[j1]], g1, s1).wait()
                pltpu.sync_copy(g1, z_sh.at[col_v.at[j1]], add=True)

                @pl.when(j1 + 2 < ch)
                def _():
                    pltpu.async_copy(y_hbm.at[row_v.at[j1 + 2]], g1, s1)

                return carry

            lax.fori_loop(0, ch // 2, body, 0)
            plsc.subcore_barrier()
            pltpu.sync_copy(
                z_sh.at[pl.ds(s * ROWS_PER_TILE, ROWS_PER_TILE)],
                out_hbm.at[c, pl.ds(s * ROWS_PER_TILE, ROWS_PER_TILE)],
            )

        pl.run_scoped(inner, pltpu.VMEM_SHARED((NPAD, h), _F32))

    return agg_k(y, rows3, cols3, zero_blk)


def _tc_prelayer(degp, x, w1):
    n = x.shape[0]
    h = w1.shape[1]

    def body(deg_ref, x_ref, w_ref, dis_ref, y_ref):
        deg = deg_ref[0] + deg_ref[1] + 1.0
        dis = lax.rsqrt(deg)
        dis_ref[...] = dis
        xw = jnp.dot(x_ref[...], w_ref[...], preferred_element_type=_F32)
        y_ref[...] = dis[0:n] * xw

    return pl.pallas_call(
        body,
        out_shape=(
            jax.ShapeDtypeStruct((NPAD, 1), _F32),
            jax.ShapeDtypeStruct((n, h), _F32),
        ),
    )(degp, x, w1)


def _tc_midlayer(zp, y, dis, b, w2):
    n, h = y.shape
    h2 = w2.shape[1]

    def body(z_ref, y_ref, dis_ref, b_ref, w_ref, out_ref):
        dis_n = dis_ref[0:n]
        agg = z_ref[0][0:n] + z_ref[1][0:n] + y_ref[...]
        hid = jax.nn.relu(dis_n * agg + b_ref[...])
        out_ref[...] = dis_n * jnp.dot(hid, w_ref[...], preferred_element_type=_F32)

    return pl.pallas_call(
        body,
        out_shape=jax.ShapeDtypeStruct((n, h2), _F32),
    )(zp, y, dis, b, w2)


def _tc_head(zp, y, dis, b, batch2d, wl, bl):
    n, h = y.shape

    def body(z_ref, y_ref, dis_ref, b_ref, batch_ref, wl_ref, bl_ref, out_ref):
        dis_n = dis_ref[0:n]
        agg = z_ref[0][0:n] + z_ref[1][0:n] + y_ref[...]
        hid = jax.nn.relu(dis_n * agg + b_ref[...])
        gids = lax.broadcasted_iota(jnp.int32, (G_GRAPHS, n), 0)
        mask = jnp.where(batch_ref[...] == gids, 1.0, 0.0).astype(_F32)
        sums = jnp.dot(mask, hid, preferred_element_type=_F32)
        cnt = jnp.sum(mask, axis=1, keepdims=True)
        pooled = sums / jnp.maximum(cnt, 1.0)
        logits = jnp.dot(pooled, wl_ref[...], preferred_element_type=_F32) + bl_ref[...]
        out_ref[...] = jax.nn.sigmoid(logits)

    return pl.pallas_call(
        body,
        out_shape=jax.ShapeDtypeStruct((G_GRAPHS, 1), _F32),
    )(zp, y, dis, b, batch2d, wl, bl)


def kernel(x, edge_index, batch, W1, b1, W2, b2, Wl, bl):
    n = x.shape[0]
    e = edge_index.shape[1]

    ch = -(-e // (NW * CHUNK))
    ch += ch % 2
    epad = NW * ch * CHUNK
    npad_edges = epad - e
    pad_i = jnp.arange(npad_edges, dtype=jnp.int32)
    pad_rows = pad_i % jnp.int32(n)
    pad_cols = jnp.int32(n) + pad_i % jnp.int32(NPAD - n)
    rows3 = jnp.concatenate([edge_index[0], pad_rows]).reshape(NW, ch, CHUNK)
    cols3 = jnp.concatenate([edge_index[1], pad_cols]).reshape(NW, ch, CHUNK)

    ones_col = jnp.ones((CHUNK, 1), _F32)
    zero_col = jnp.zeros((ROWS_PER_TILE, 1), _F32)
    zero_blk = jnp.zeros((ROWS_PER_TILE, W1.shape[1]), _F32)

    degp = _sc_degree(cols3, ones_col, zero_col)
    dis, y1 = _tc_prelayer(degp, x, W1)
    z1 = _sc_aggregate(y1, rows3, cols3, zero_blk)
    y2 = _tc_midlayer(z1, y1, dis, b1.reshape(1, -1), W2)
    z2 = _sc_aggregate(y2, rows3, cols3, zero_blk)
    return _tc_head(
        z2, y2, dis, b2.reshape(1, -1), batch.reshape(1, n).astype(jnp.int32),
        Wl, bl.reshape(1, 1),
    )


# speedup vs baseline: 39.0055x; 39.0055x over previous
import functools

import jax
import jax.numpy as jnp
from jax import lax
from jax.experimental import pallas as pl
from jax.experimental.pallas import tpu as pltpu
from jax.experimental.pallas import tpu_sc as plsc

G_GRAPHS = 64
NC = 2
NS = 16
NW = NC * NS
CHUNK = 128
ROWS_PER_TILE = 632
NPAD = NS * ROWS_PER_TILE

_F32 = jnp.float32


def _mesh():
    return plsc.VectorSubcoreMesh(core_axis_name="c", subcore_axis_name="s")


DEGW = 16


def _sc_degree(cols3, ones_col, zero_col):
    ch = cols3.shape[1]

    @functools.partial(
        pl.kernel,
        mesh=_mesh(),
        out_type=jax.ShapeDtypeStruct((NC, NPAD, DEGW), _F32),
        compiler_params=pltpu.CompilerParams(use_tc_tiling_on_sc=False),
        scratch_types=[
            pltpu.VMEM((ch, CHUNK), jnp.int32),
            pltpu.VMEM((CHUNK, DEGW), _F32),
            pltpu.VMEM_SHARED((NPAD, DEGW), _F32),
        ],
    )
    def deg_k(col_hbm, ones_hbm, zero_hbm, out_hbm, col_v, ones_v, deg_sh):
        c = lax.axis_index("c")
        s = lax.axis_index("s")
        w = c * NS + s
        pltpu.sync_copy(col_hbm.at[w], col_v)
        pltpu.sync_copy(ones_hbm, ones_v)
        pltpu.sync_copy(zero_hbm, deg_sh.at[pl.ds(s * ROWS_PER_TILE, ROWS_PER_TILE)])
        plsc.subcore_barrier()

        def body(j, carry):
            pltpu.sync_copy(ones_v, deg_sh.at[col_v.at[j]], add=True)
            return carry

        lax.fori_loop(0, ch, body, 0)
        plsc.subcore_barrier()
        pltpu.sync_copy(
            deg_sh.at[pl.ds(s * ROWS_PER_TILE, ROWS_PER_TILE)],
            out_hbm.at[c, pl.ds(s * ROWS_PER_TILE, ROWS_PER_TILE)],
        )

    return deg_k(cols3, ones_col, zero_col)


def _sc_aggregate(y, rows3, cols3, zero_blk):
    h = y.shape[1]
    ch = rows3.shape[1]

    @functools.partial(
        pl.kernel,
        mesh=_mesh(),
        out_type=jax.ShapeDtypeStruct((NC, NPAD, h), _F32),
        compiler_params=pltpu.CompilerParams(use_tc_tiling_on_sc=False),
        scratch_types=[
            pltpu.VMEM((ch, CHUNK), jnp.int32),
            pltpu.VMEM((ch, CHUNK), jnp.int32),
            pltpu.VMEM((CHUNK, h), _F32),
            pltpu.VMEM((CHUNK, h), _F32),
            pltpu.VMEM_SHARED((NPAD, h), _F32),
            pltpu.SemaphoreType.DMA,
            pltpu.SemaphoreType.DMA,
        ],
    )
    def agg_k(y_hbm, row_hbm, col_hbm, zero_hbm, out_hbm,
              row_v, col_v, g0, g1, z_sh, s0, s1):
        c = lax.axis_index("c")
        s = lax.axis_index("s")
        w = c * NS + s
        pltpu.sync_copy(row_hbm.at[w], row_v)
        pltpu.sync_copy(col_hbm.at[w], col_v)
        pltpu.sync_copy(zero_hbm, z_sh.at[pl.ds(s * ROWS_PER_TILE, ROWS_PER_TILE)])
        plsc.subcore_barrier()

        pltpu.async_copy(y_hbm.at[row_v.at[0]], g0, s0)
        pltpu.async_copy(y_hbm.at[row_v.at[1]], g1, s1)

        def body(i, carry):
            j0 = 2 * i
            j1 = j0 + 1
            pltpu.make_async_copy(y_hbm.at[row_v.at[j0]], g0, s0).wait()
            pltpu.sync_copy(g0, z_sh.at[col_v.at[j0]], add=True)

            @pl.when(j0 + 2 < ch)
            def _():
                pltpu.async_copy(y_hbm.at[row_v.at[j0 + 2]], g0, s0)

            pltpu.make_async_copy(y_hbm.at[row_v.at[j1]], g1, s1).wait()
            pltpu.sync_copy(g1, z_sh.at[col_v.at[j1]], add=True)

            @pl.when(j1 + 2 < ch)
            def _():
                pltpu.async_copy(y_hbm.at[row_v.at[j1 + 2]], g1, s1)

            return carry

        lax.fori_loop(0, ch // 2, body, 0)
        plsc.subcore_barrier()
        pltpu.sync_copy(
            z_sh.at[pl.ds(s * ROWS_PER_TILE, ROWS_PER_TILE)],
            out_hbm.at[c, pl.ds(s * ROWS_PER_TILE, ROWS_PER_TILE)],
        )

    return agg_k(y, rows3, cols3, zero_blk)


def _tc_prelayer(degp, x, w1):
    n = x.shape[0]
    h = w1.shape[1]

    def body(deg_ref, x_ref, w_ref, dis_ref, y_ref):
        deg = deg_ref[0][:, 0:1] + deg_ref[1][:, 0:1] + 1.0
        dis = lax.rsqrt(deg)
        dis_ref[...] = dis
        xw = jnp.dot(x_ref[...], w_ref[...], preferred_element_type=_F32)
        y_ref[...] = dis[0:n] * xw

    return pl.pallas_call(
        body,
        out_shape=(
            jax.ShapeDtypeStruct((NPAD, 1), _F32),
            jax.ShapeDtypeStruct((n, h), _F32),
        ),
    )(degp, x, w1)


def _tc_midlayer(zp, y, dis, b, w2):
    n, h = y.shape
    h2 = w2.shape[1]

    def body(z_ref, y_ref, dis_ref, b_ref, w_ref, out_ref):
        dis_n = dis_ref[0:n]
        agg = z_ref[0][0:n] + z_ref[1][0:n] + y_ref[...]
        hid = jax.nn.relu(dis_n * agg + b_ref[...])
        out_ref[...] = dis_n * jnp.dot(hid, w_ref[...], preferred_element_type=_F32)

    return pl.pallas_call(
        body,
        out_shape=jax.ShapeDtypeStruct((n, h2), _F32),
    )(zp, y, dis, b, w2)


def _tc_head(zp, y, dis, b, batch2d, wl, bl):
    n, h = y.shape

    def body(z_ref, y_ref, dis_ref, b_ref, batch_ref, wl_ref, bl_ref, out_ref):
        dis_n = dis_ref[0:n]
        agg = z_ref[0][0:n] + z_ref[1][0:n] + y_ref[...]
        hid = jax.nn.relu(dis_n * agg + b_ref[...])
        gids = lax.broadcasted_iota(jnp.int32, (G_GRAPHS, n), 0)
        mask = jnp.where(batch_ref[...] == gids, 1.0, 0.0).astype(_F32)
        sums = jnp.dot(mask, hid, preferred_element_type=_F32)
        cnt = jnp.sum(mask, axis=1, keepdims=True)
        pooled = sums / jnp.maximum(cnt, 1.0)
        logits = jnp.dot(pooled, wl_ref[...], preferred_element_type=_F32) + bl_ref[...]
        out_ref[...] = jax.nn.sigmoid(logits)

    return pl.pallas_call(
        body,
        out_shape=jax.ShapeDtypeStruct((G_GRAPHS, 1), _F32),
    )(zp, y, dis, b, batch2d, wl, bl)


def kernel(x, edge_index, batch, W1, b1, W2, b2, Wl, bl):
    n = x.shape[0]
    e = edge_index.shape[1]

    ch = -(-e // (NW * CHUNK))
    ch += ch % 2
    epad = NW * ch * CHUNK
    npad_edges = epad - e
    pad_i = jnp.arange(npad_edges, dtype=jnp.int32)
    pad_rows = pad_i % jnp.int32(n)
    pad_cols = jnp.int32(n) + pad_i % jnp.int32(NPAD - n)
    rows3 = jnp.concatenate([edge_index[0], pad_rows]).reshape(NW, ch, CHUNK)
    cols3 = jnp.concatenate([edge_index[1], pad_cols]).reshape(NW, ch, CHUNK)

    ones_col = jnp.ones((CHUNK, DEGW), _F32)
    zero_col = jnp.zeros((ROWS_PER_TILE, DEGW), _F32)
    zero_blk = jnp.zeros((ROWS_PER_TILE, W1.shape[1]), _F32)

    degp = _sc_degree(cols3, ones_col, zero_col)
    dis, y1 = _tc_prelayer(degp, x, W1)
    z1 = _sc_aggregate(y1, rows3, cols3, zero_blk)
    y2 = _tc_midlayer(z1, y1, dis, b1.reshape(1, -1), W2)
    z2 = _sc_aggregate(y2, rows3, cols3, zero_blk)
    return _tc_head(
        z2, y2, dis, b2.reshape(1, -1), batch.reshape(1, n).astype(jnp.int32),
        Wl, bl.reshape(1, 1),
    )


# speedup vs baseline: 44.3475x; 1.1370x over previous
import functools

import jax
import jax.numpy as jnp
from jax import lax
from jax.experimental import pallas as pl
from jax.experimental.pallas import tpu as pltpu
from jax.experimental.pallas import tpu_sc as plsc

G_GRAPHS = 64
NC = 2
NS = 16
NW = NC * NS
CHUNK = 128
ROWS_PER_TILE = 632
NPAD = NS * ROWS_PER_TILE

_F32 = jnp.float32


def _mesh():
    return plsc.VectorSubcoreMesh(core_axis_name="c", subcore_axis_name="s")


DEGW = 16


def _sc_degree(cols3, ones_col, zero_col):
    ch = cols3.shape[1]

    @functools.partial(
        pl.kernel,
        mesh=_mesh(),
        out_type=jax.ShapeDtypeStruct((NC, NPAD, DEGW), _F32),
        compiler_params=pltpu.CompilerParams(use_tc_tiling_on_sc=False),
        scratch_types=[
            pltpu.VMEM((ch, CHUNK), jnp.int32),
            pltpu.VMEM((CHUNK, DEGW), _F32),
            pltpu.VMEM_SHARED((NPAD, DEGW), _F32),
        ],
    )
    def deg_k(col_hbm, ones_hbm, zero_hbm, out_hbm, col_v, ones_v, deg_sh):
        c = lax.axis_index("c")
        s = lax.axis_index("s")
        w = c * NS + s
        pltpu.sync_copy(col_hbm.at[w], col_v)
        pltpu.sync_copy(ones_hbm, ones_v)
        pltpu.sync_copy(zero_hbm, deg_sh.at[pl.ds(s * ROWS_PER_TILE, ROWS_PER_TILE)])
        plsc.subcore_barrier()

        def body(j, carry):
            pltpu.sync_copy(ones_v, deg_sh.at[col_v.at[j]], add=True)
            return carry

        lax.fori_loop(0, ch, body, 0)
        plsc.subcore_barrier()
        pltpu.sync_copy(
            deg_sh.at[pl.ds(s * ROWS_PER_TILE, ROWS_PER_TILE)],
            out_hbm.at[c, pl.ds(s * ROWS_PER_TILE, ROWS_PER_TILE)],
        )

    return deg_k(cols3, ones_col, zero_col)


def _sc_aggregate(y, rows3, cols3, zero_blk):
    h = y.shape[1]
    ch = rows3.shape[1]

    @functools.partial(
        pl.kernel,
        mesh=_mesh(),
        out_type=jax.ShapeDtypeStruct((NC, NPAD, h), _F32),
        compiler_params=pltpu.CompilerParams(use_tc_tiling_on_sc=False),
        scratch_types=[
            pltpu.VMEM((ch, CHUNK), jnp.int32),
            pltpu.VMEM((ch, CHUNK), jnp.int32),
            pltpu.VMEM((CHUNK, h), _F32),
            pltpu.VMEM((CHUNK, h), _F32),
            pltpu.VMEM((CHUNK, h), _F32),
            pltpu.VMEM((CHUNK, h), _F32),
            pltpu.VMEM_SHARED((NPAD, h), _F32),
            pltpu.SemaphoreType.DMA,
            pltpu.SemaphoreType.DMA,
            pltpu.SemaphoreType.DMA,
            pltpu.SemaphoreType.DMA,
        ],
    )
    def agg_k(y_hbm, row_hbm, col_hbm, zero_hbm, out_hbm,
              row_v, col_v, g0, g1, g2, g3, z_sh, s0, s1, s2, s3):
        c = lax.axis_index("c")
        s = lax.axis_index("s")
        w = c * NS + s
        pltpu.sync_copy(row_hbm.at[w], row_v)
        pltpu.sync_copy(col_hbm.at[w], col_v)
        pltpu.sync_copy(zero_hbm, z_sh.at[pl.ds(s * ROWS_PER_TILE, ROWS_PER_TILE)])
        plsc.subcore_barrier()

        bufs = (g0, g1, g2, g3)
        sems = (s0, s1, s2, s3)
        nb = len(bufs)

        for b in range(nb):
            pltpu.async_copy(y_hbm.at[row_v.at[b]], bufs[b], sems[b])

        def body(i, carry):
            for b in range(nb):
                j = nb * i + b
                pltpu.make_async_copy(y_hbm.at[row_v.at[j]], bufs[b], sems[b]).wait()
                pltpu.sync_copy(bufs[b], z_sh.at[col_v.at[j]], add=True)

                @pl.when(j + nb < ch)
                def _():
                    pltpu.async_copy(y_hbm.at[row_v.at[j + nb]], bufs[b], sems[b])

            return carry

        lax.fori_loop(0, ch // nb, body, 0)
        plsc.subcore_barrier()
        pltpu.sync_copy(
            z_sh.at[pl.ds(s * ROWS_PER_TILE, ROWS_PER_TILE)],
            out_hbm.at[c, pl.ds(s * ROWS_PER_TILE, ROWS_PER_TILE)],
        )

    return agg_k(y, rows3, cols3, zero_blk)


def _tc_prelayer(degp, x, w1):
    n = x.shape[0]
    h = w1.shape[1]

    def body(deg_ref, x_ref, w_ref, dis_ref, y_ref):
        deg = deg_ref[0][:, 0:1] + deg_ref[1][:, 0:1] + 1.0
        dis = lax.rsqrt(deg)
        dis_ref[...] = dis
        xw = jnp.dot(x_ref[...], w_ref[...], preferred_element_type=_F32)
        y_ref[...] = dis[0:n] * xw

    return pl.pallas_call(
        body,
        out_shape=(
            jax.ShapeDtypeStruct((NPAD, 1), _F32),
            jax.ShapeDtypeStruct((n, h), _F32),
        ),
    )(degp, x, w1)


def _tc_midlayer(zp, y, dis, b, w2):
    n, h = y.shape
    h2 = w2.shape[1]

    def body(z_ref, y_ref, dis_ref, b_ref, w_ref, out_ref):
        dis_n = dis_ref[0:n]
        agg = z_ref[0][0:n] + z_ref[1][0:n] + y_ref[...]
        hid = jax.nn.relu(dis_n * agg + b_ref[...])
        out_ref[...] = dis_n * jnp.dot(hid, w_ref[...], preferred_element_type=_F32)

    return pl.pallas_call(
        body,
        out_shape=jax.ShapeDtypeStruct((n, h2), _F32),
    )(zp, y, dis, b, w2)


def _tc_head(zp, y, dis, b, batch2d, wl, bl):
    n, h = y.shape

    def body(z_ref, y_ref, dis_ref, b_ref, batch_ref, wl_ref, bl_ref, out_ref):
        dis_n = dis_ref[0:n]
        agg = z_ref[0][0:n] + z_ref[1][0:n] + y_ref[...]
        hid = jax.nn.relu(dis_n * agg + b_ref[...])
        gids = lax.broadcasted_iota(jnp.int32, (G_GRAPHS, n), 0)
        mask = jnp.where(batch_ref[...] == gids, 1.0, 0.0).astype(_F32)
        sums = jnp.dot(mask, hid, preferred_element_type=_F32)
        cnt = jnp.sum(mask, axis=1, keepdims=True)
        pooled = sums / jnp.maximum(cnt, 1.0)
        logits = jnp.dot(pooled, wl_ref[...], preferred_element_type=_F32) + bl_ref[...]
        out_ref[...] = jax.nn.sigmoid(logits)

    return pl.pallas_call(
        body,
        out_shape=jax.ShapeDtypeStruct((G_GRAPHS, 1), _F32),
    )(zp, y, dis, b, batch2d, wl, bl)


def kernel(x, edge_index, batch, W1, b1, W2, b2, Wl, bl):
    n = x.shape[0]
    e = edge_index.shape[1]

    ch = -(-e // (NW * CHUNK))
    ch = -(-ch // 4) * 4
    epad = NW * ch * CHUNK
    npad_edges = epad - e
    pad_i = jnp.arange(npad_edges, dtype=jnp.int32)
    pad_rows = pad_i % jnp.int32(n)
    pad_cols = jnp.int32(n) + pad_i % jnp.int32(NPAD - n)
    rows3 = jnp.concatenate([edge_index[0], pad_rows]).reshape(NW, ch, CHUNK)
    cols3 = jnp.concatenate([edge_index[1], pad_cols]).reshape(NW, ch, CHUNK)

    ones_col = jnp.ones((CHUNK, DEGW), _F32)
    zero_col = jnp.zeros((ROWS_PER_TILE, DEGW), _F32)
    zero_blk = jnp.zeros((ROWS_PER_TILE, W1.shape[1]), _F32)

    degp = _sc_degree(cols3, ones_col, zero_col)
    dis, y1 = _tc_prelayer(degp, x, W1)
    z1 = _sc_aggregate(y1, rows3, cols3, zero_blk)
    y2 = _tc_midlayer(z1, y1, dis, b1.reshape(1, -1), W2)
    z2 = _sc_aggregate(y2, rows3, cols3, zero_blk)
    return _tc_head(
        z2, y2, dis, b2.reshape(1, -1), batch.reshape(1, n).astype(jnp.int32),
        Wl, bl.reshape(1, 1),
    )
